# Initial kernel scaffold; baseline (speedup 1.0000x reference)
import jax
import jax.numpy as jnp
from jax.experimental import pallas as pl


def kernel(x, edge_index, W_l, b_l, W_r, gn_weight, gn_bias, gn_mean_scale):
    raise NotImplementedError("write your pallas kernel here")


# speedup vs baseline: 5.1265x; 5.1265x over previous
import functools

import jax
import jax.numpy as jnp
from jax import lax
from jax.experimental import pallas as pl
from jax.experimental.pallas import tpu as pltpu
from jax.experimental.pallas import tpu_sc as plsc

N = 10000
D = 128
E = 320000
NC = 2
NS = 16
NW = NC * NS
CH = 50
E_W = E // NW
CHUNKS = E_W // CH
G = 8
GRP = CHUNKS // G
ROW_STRIDE = 624
ROWS_T = 640
PZ = 40
NPC = ROWS_T // PZ


def _sc_gather_scatter(x, src_r, dst_r, zeros_b):
    mesh = plsc.VectorSubcoreMesh(core_axis_name="c", subcore_axis_name="s")

    @functools.partial(
        pl.kernel,
        out_type=jax.ShapeDtypeStruct((NC, N, D), jnp.float32),
        mesh=mesh,
        scratch_types=(
            pltpu.VMEM((G, CH), jnp.int32),
            pltpu.VMEM((G, CH), jnp.int32),
            pltpu.VMEM((CH, D), jnp.float32),
            pltpu.VMEM((PZ, D), jnp.float32),
            pltpu.VMEM_SHARED((N, D), jnp.float32),
        ),
    )
    def sc_fn(x_hbm, src_hbm, dst_hbm, z_hbm, agg_out,
              src_v, dst_v, buf, bx, agg_sp):
        c = lax.axis_index("c")
        s = lax.axis_index("s")
        wid = c * NS + s
        r0 = s * ROW_STRIDE
        pltpu.sync_copy(z_hbm, bx)

        @pl.loop(0, NPC)
        def _init(t):
            pltpu.sync_copy(bx, agg_sp.at[pl.ds(r0 + t * PZ, PZ)])

        plsc.subcore_barrier()

        @pl.loop(0, GRP)
        def _group(k):
            pltpu.sync_copy(src_hbm.at[wid, k], src_v)
            pltpu.sync_copy(dst_hbm.at[wid, k], dst_v)
            for b in range(G):
                pltpu.sync_copy(x_hbm.at[src_v.at[b]], buf)
                pltpu.sync_copy(buf, agg_sp.at[dst_v.at[b]], add=True)

        plsc.subcore_barrier()

        @pl.loop(0, NPC)
        def _copyout(t):
            pc = pl.ds(r0 + t * PZ, PZ)
            pltpu.sync_copy(agg_sp.at[pc], bx)
            pltpu.sync_copy(bx, agg_out.at[c, pc])

    return sc_fn(x, src_r, dst_r, zeros_b)


def _sc_degree(dst_r, ones_b, zeros_b):
    mesh = plsc.VectorSubcoreMesh(core_axis_name="c", subcore_axis_name="s")

    @functools.partial(
        pl.kernel,
        out_type=jax.ShapeDtypeStruct((NC, N, D), jnp.float32),
        mesh=mesh,
        scratch_types=(
            pltpu.VMEM((G, CH), jnp.int32),
            pltpu.VMEM((CH, D), jnp.float32),
            pltpu.VMEM((PZ, D), jnp.float32),
            pltpu.VMEM_SHARED((N, D), jnp.float32),
        ),
    )
    def sc_fn(dst_hbm, ones_hbm, z_hbm, deg_out, dst_v, ones_v, bx, deg_sp):
        c = lax.axis_index("c")
        s = lax.axis_index("s")
        wid = c * NS + s
        r0 = s * ROW_STRIDE
        pltpu.sync_copy(ones_hbm, ones_v)
        pltpu.sync_copy(z_hbm, bx)

        @pl.loop(0, NPC)
        def _init(t):
            pltpu.sync_copy(bx, deg_sp.at[pl.ds(r0 + t * PZ, PZ)])

        plsc.subcore_barrier()

        @pl.loop(0, GRP)
        def _group(k):
            pltpu.sync_copy(dst_hbm.at[wid, k], dst_v)
            for b in range(G):
                pltpu.sync_copy(ones_v, deg_sp.at[dst_v.at[b]], add=True)

        plsc.subcore_barrier()

        @pl.loop(0, NPC)
        def _copyout(t):
            pc = pl.ds(r0 + t * PZ, PZ)
            pltpu.sync_copy(deg_sp.at[pc], bx)
            pltpu.sync_copy(bx, deg_out.at[c, pc])

    return sc_fn(dst_r, ones_b, zeros_b)


BLK = 1000
NBLK = N // BLK


def _tc_stage1(x_ref, agg_ref, deg_ref, wl_ref, bl_ref, wr_ref,
               h_ref, stats_ref):
    x = x_ref[...]
    agg = agg_ref[0] + agg_ref[1]
    deg = deg_ref[0, :, 0] + deg_ref[1, :, 0]
    inv = 1.0 / jnp.maximum(deg, 1.0)
    mean_agg = agg * inv[:, None]
    hi = jax.lax.Precision.HIGHEST
    h = lax.dot_general(mean_agg, wl_ref[...], (((1,), (1,)), ((), ())),
                        precision=hi, preferred_element_type=jnp.float32)
    h = h + lax.dot_general(x, wr_ref[...], (((1,), (1,)), ((), ())),
                            precision=hi, preferred_element_type=jnp.float32)
    h = h + bl_ref[...][None, :] + x
    h = jnp.maximum(h, 0.0)
    h_ref[...] = h
    s1 = jnp.sum(h, axis=0, keepdims=True)
    s2 = jnp.sum(h * h, axis=0, keepdims=True)
    stats_ref[...] = jnp.concatenate([s1, s2, jnp.zeros((6, D), jnp.float32)],
                                     axis=0)[None]


def _tc_stage2(h_ref, stats_ref, gw_ref, gb_ref, gms_ref, out_ref):
    stats = stats_ref[...]
    m = jnp.sum(stats[:, 0, :], axis=0) / N
    q = jnp.sum(stats[:, 1, :], axis=0) / N
    ms = gms_ref[...]
    var = q + m * m * ms * (ms - 2.0)
    scale = gw_ref[...] * lax.rsqrt(var + 1e-5)
    shift = gb_ref[...] - m * ms * scale
    out_ref[...] = h_ref[...] * scale[None, :] + shift[None, :]


def kernel(x, edge_index, W_l, b_l, W_r, gn_weight, gn_bias, gn_mean_scale):
    src_r = edge_index[0].reshape(NW, GRP, G, CH)
    dst_r = edge_index[1].reshape(NW, GRP, G, CH)
    zeros_b = jnp.zeros((PZ, D), jnp.float32)
    ones_b = jnp.ones((CH, D), jnp.float32)
    agg_parts = _sc_gather_scatter(x, src_r, dst_r, zeros_b)
    deg_parts = _sc_degree(dst_r, ones_b, zeros_b)
    h, stats = pl.pallas_call(
        _tc_stage1,
        grid=(NBLK,),
        in_specs=[
            pl.BlockSpec((BLK, D), lambda i: (i, 0)),
            pl.BlockSpec((NC, BLK, D), lambda i: (0, i, 0)),
            pl.BlockSpec((NC, BLK, D), lambda i: (0, i, 0)),
            pl.BlockSpec((D, D), lambda i: (0, 0)),
            pl.BlockSpec((D,), lambda i: (0,)),
            pl.BlockSpec((D, D), lambda i: (0, 0)),
        ],
        out_specs=[
            pl.BlockSpec((BLK, D), lambda i: (i, 0)),
            pl.BlockSpec((1, 8, D), lambda i: (i, 0, 0)),
        ],
        out_shape=[
            jax.ShapeDtypeStruct((N, D), jnp.float32),
            jax.ShapeDtypeStruct((NBLK, 8, D), jnp.float32),
        ],
    )(x, agg_parts, deg_parts, W_l, b_l, W_r)
    return pl.pallas_call(
        _tc_stage2,
        grid=(NBLK,),
        in_specs=[
            pl.BlockSpec((BLK, D), lambda i: (i, 0)),
            pl.BlockSpec((NBLK, 8, D), lambda i: (0, 0, 0)),
            pl.BlockSpec((D,), lambda i: (0,)),
            pl.BlockSpec((D,), lambda i: (0,)),
            pl.BlockSpec((D,), lambda i: (0,)),
        ],
        out_specs=pl.BlockSpec((BLK, D), lambda i: (i, 0)),
        out_shape=jax.ShapeDtypeStruct((N, D), jnp.float32),
    )(h, stats, gn_weight, gn_bias, gn_mean_scale)


# speedup vs baseline: 6.3487x; 1.2384x over previous
import functools

import jax
import jax.numpy as jnp
from jax import lax
from jax.experimental import pallas as pl
from jax.experimental.pallas import tpu as pltpu
from jax.experimental.pallas import tpu_sc as plsc

N = 10000
D = 128
E = 320000
NC = 2
NS = 16
NW = NC * NS
CH = 50
E_W = E // NW
CHUNKS = E_W // CH
G = 4
GRP = CHUNKS // G
G2 = 10
CH2 = 100
GRP2 = E_W // (G2 * CH2)
ROW_STRIDE = 624
ROWS_T = 640
PZ = 40
NPC = ROWS_T // PZ


def _sc_gather_scatter(x, src_r, dst_r, zeros_b):
    mesh = plsc.VectorSubcoreMesh(core_axis_name="c", subcore_axis_name="s")

    @functools.partial(
        pl.kernel,
        out_type=jax.ShapeDtypeStruct((NC, N, D), jnp.float32),
        mesh=mesh,
        scratch_types=(
            pltpu.VMEM((G, CH), jnp.int32),
            pltpu.VMEM((G, CH), jnp.int32),
            pltpu.VMEM((CH, D), jnp.float32),
            pltpu.VMEM((CH, D), jnp.float32),
            pltpu.VMEM((CH, D), jnp.float32),
            pltpu.VMEM((CH, D), jnp.float32),
            pltpu.VMEM((PZ, D), jnp.float32),
            pltpu.VMEM_SHARED((N, D), jnp.float32),
            pltpu.SemaphoreType.DMA,
            pltpu.SemaphoreType.DMA,
            pltpu.SemaphoreType.DMA,
            pltpu.SemaphoreType.DMA,
            pltpu.SemaphoreType.DMA,
            pltpu.SemaphoreType.DMA,
            pltpu.SemaphoreType.DMA,
            pltpu.SemaphoreType.DMA,
        ),
    )
    def sc_fn(x_hbm, src_hbm, dst_hbm, z_hbm, agg_out,
              src_v, dst_v, b0, b1, b2, b3, bx, agg_sp,
              g0, g1, g2, g3, s0, s1, s2, s3):
        bufs = (b0, b1, b2, b3)
        gsems = (g0, g1, g2, g3)
        ssems = (s0, s1, s2, s3)
        c = lax.axis_index("c")
        s = lax.axis_index("s")
        wid = c * NS + s
        r0 = s * ROW_STRIDE
        pltpu.sync_copy(z_hbm, bx)

        @pl.loop(0, NPC)
        def _init(t):
            pltpu.sync_copy(bx, agg_sp.at[pl.ds(r0 + t * PZ, PZ)])

        plsc.subcore_barrier()

        @pl.loop(0, GRP)
        def _group(k):
            pltpu.sync_copy(src_hbm.at[wid, k], src_v)
            pltpu.sync_copy(dst_hbm.at[wid, k], dst_v)
            gds = [pltpu.async_copy(x_hbm.at[src_v.at[b]], bufs[b], gsems[b])
                   for b in range(G)]
            sds = []
            for b in range(G):
                gds[b].wait()
                sds.append(pltpu.async_copy(bufs[b], agg_sp.at[dst_v.at[b]],
                                            ssems[b], add=True))
            for d in sds:
                d.wait()

        plsc.subcore_barrier()

        @pl.loop(0, NPC)
        def _copyout(t):
            pc = pl.ds(r0 + t * PZ, PZ)
            pltpu.sync_copy(agg_sp.at[pc], bx)
            pltpu.sync_copy(bx, agg_out.at[c, pc])

    return sc_fn(x, src_r, dst_r, zeros_b)


def _sc_degree(dst_r, ones_b, zeros_b):
    mesh = plsc.VectorSubcoreMesh(core_axis_name="c", subcore_axis_name="s")

    @functools.partial(
        pl.kernel,
        out_type=jax.ShapeDtypeStruct((NC, N, D), jnp.float32),
        mesh=mesh,
        scratch_types=(
            pltpu.VMEM((G2, CH2), jnp.int32),
            pltpu.VMEM((CH2, D), jnp.float32),
            pltpu.VMEM((PZ, D), jnp.float32),
            pltpu.VMEM_SHARED((N, D), jnp.float32),
            pltpu.SemaphoreType.DMA,
        ),
    )
    def sc_fn(dst_hbm, ones_hbm, z_hbm, deg_out, dst_v, ones_v, bx, deg_sp,
              ssem):
        c = lax.axis_index("c")
        s = lax.axis_index("s")
        wid = c * NS + s
        r0 = s * ROW_STRIDE
        pltpu.sync_copy(ones_hbm, ones_v)
        pltpu.sync_copy(z_hbm, bx)

        @pl.loop(0, NPC)
        def _init(t):
            pltpu.sync_copy(bx, deg_sp.at[pl.ds(r0 + t * PZ, PZ)])

        plsc.subcore_barrier()

        @pl.loop(0, GRP2)
        def _group(k):
            pltpu.sync_copy(dst_hbm.at[wid, k], dst_v)
            sds = [pltpu.async_copy(ones_v, deg_sp.at[dst_v.at[b]], ssem,
                                    add=True)
                   for b in range(G2)]
            for d in sds:
                d.wait()

        plsc.subcore_barrier()

        @pl.loop(0, NPC)
        def _copyout(t):
            pc = pl.ds(r0 + t * PZ, PZ)
            pltpu.sync_copy(deg_sp.at[pc], bx)
            pltpu.sync_copy(bx, deg_out.at[c, pc])

    return sc_fn(dst_r, ones_b, zeros_b)


BLK = 1000
NBLK = N // BLK


def _tc_stage1(x_ref, agg_ref, deg_ref, wl_ref, bl_ref, wr_ref,
               h_ref, stats_ref):
    x = x_ref[...]
    agg = agg_ref[0] + agg_ref[1]
    deg = deg_ref[0, :, 0] + deg_ref[1, :, 0]
    inv = 1.0 / jnp.maximum(deg, 1.0)
    mean_agg = agg * inv[:, None]
    hi = jax.lax.Precision.HIGHEST
    h = lax.dot_general(mean_agg, wl_ref[...], (((1,), (1,)), ((), ())),
                        precision=hi, preferred_element_type=jnp.float32)
    h = h + lax.dot_general(x, wr_ref[...], (((1,), (1,)), ((), ())),
                            precision=hi, preferred_element_type=jnp.float32)
    h = h + bl_ref[...][None, :] + x
    h = jnp.maximum(h, 0.0)
    h_ref[...] = h
    s1 = jnp.sum(h, axis=0, keepdims=True)
    s2 = jnp.sum(h * h, axis=0, keepdims=True)
    stats_ref[...] = jnp.concatenate([s1, s2, jnp.zeros((6, D), jnp.float32)],
                                     axis=0)[None]


def _tc_stage2(h_ref, stats_ref, gw_ref, gb_ref, gms_ref, out_ref):
    stats = stats_ref[...]
    m = jnp.sum(stats[:, 0, :], axis=0) / N
    q = jnp.sum(stats[:, 1, :], axis=0) / N
    ms = gms_ref[...]
    var = q + m * m * ms * (ms - 2.0)
    scale = gw_ref[...] * lax.rsqrt(var + 1e-5)
    shift = gb_ref[...] - m * ms * scale
    out_ref[...] = h_ref[...] * scale[None, :] + shift[None, :]


def kernel(x, edge_index, W_l, b_l, W_r, gn_weight, gn_bias, gn_mean_scale):
    src_r = edge_index[0].reshape(NW, GRP, G, CH)
    dst_r = edge_index[1].reshape(NW, GRP, G, CH)
    dst_r2 = edge_index[1].reshape(NW, GRP2, G2, CH2)
    zeros_b = jnp.zeros((PZ, D), jnp.float32)
    ones_b = jnp.ones((CH2, D), jnp.float32)
    agg_parts = _sc_gather_scatter(x, src_r, dst_r, zeros_b)
    deg_parts = _sc_degree(dst_r2, ones_b, zeros_b)
    h, stats = pl.pallas_call(
        _tc_stage1,
        grid=(NBLK,),
        in_specs=[
            pl.BlockSpec((BLK, D), lambda i: (i, 0)),
            pl.BlockSpec((NC, BLK, D), lambda i: (0, i, 0)),
            pl.BlockSpec((NC, BLK, D), lambda i: (0, i, 0)),
            pl.BlockSpec((D, D), lambda i: (0, 0)),
            pl.BlockSpec((D,), lambda i: (0,)),
            pl.BlockSpec((D, D), lambda i: (0, 0)),
        ],
        out_specs=[
            pl.BlockSpec((BLK, D), lambda i: (i, 0)),
            pl.BlockSpec((1, 8, D), lambda i: (i, 0, 0)),
        ],
        out_shape=[
            jax.ShapeDtypeStruct((N, D), jnp.float32),
            jax.ShapeDtypeStruct((NBLK, 8, D), jnp.float32),
        ],
    )(x, agg_parts, deg_parts, W_l, b_l, W_r)
    return pl.pallas_call(
        _tc_stage2,
        grid=(NBLK,),
        in_specs=[
            pl.BlockSpec((BLK, D), lambda i: (i, 0)),
            pl.BlockSpec((NBLK, 8, D), lambda i: (0, 0, 0)),
            pl.BlockSpec((D,), lambda i: (0,)),
            pl.BlockSpec((D,), lambda i: (0,)),
            pl.BlockSpec((D,), lambda i: (0,)),
        ],
        out_specs=pl.BlockSpec((BLK, D), lambda i: (i, 0)),
        out_shape=jax.ShapeDtypeStruct((N, D), jnp.float32),
    )(h, stats, gn_weight, gn_bias, gn_mean_scale)


# speedup vs baseline: 8.5047x; 1.3396x over previous
import functools

import jax
import jax.numpy as jnp
from jax import lax
from jax.experimental import pallas as pl
from jax.experimental.pallas import tpu as pltpu
from jax.experimental.pallas import tpu_sc as plsc

N = 10000
D = 128
E = 320000
NC = 2
NS = 16
NW = NC * NS
CH = 50
E_W = E // NW
CHUNKS = E_W // CH
G = 4
GRP = CHUNKS // G
G2 = 10
CH2 = 100
GRP2 = E_W // (G2 * CH2)
ROW_STRIDE = 624
ROWS_T = 640
PZ = 40
NPC = ROWS_T // PZ


def _sc_gather_scatter(x, src_r, dst_r, zeros_b):
    mesh = plsc.VectorSubcoreMesh(core_axis_name="c", subcore_axis_name="s")

    @functools.partial(
        pl.kernel,
        out_type=jax.ShapeDtypeStruct((NC, N, D), jnp.float32),
        mesh=mesh,
        scratch_types=(
            pltpu.VMEM((G, CH), jnp.int32),
            pltpu.VMEM((G, CH), jnp.int32),
            pltpu.VMEM((G, CH), jnp.int32),
            pltpu.VMEM((G, CH), jnp.int32),
            pltpu.VMEM((CH, D), jnp.float32),
            pltpu.VMEM((CH, D), jnp.float32),
            pltpu.VMEM((CH, D), jnp.float32),
            pltpu.VMEM((CH, D), jnp.float32),
            pltpu.VMEM((PZ, D), jnp.float32),
            pltpu.VMEM_SHARED((N, D), jnp.float32),
            pltpu.SemaphoreType.DMA,
            pltpu.SemaphoreType.DMA,
            pltpu.SemaphoreType.DMA,
            pltpu.SemaphoreType.DMA,
            pltpu.SemaphoreType.DMA,
            pltpu.SemaphoreType.DMA,
            pltpu.SemaphoreType.DMA,
            pltpu.SemaphoreType.DMA,
            pltpu.SemaphoreType.DMA,
            pltpu.SemaphoreType.DMA,
            pltpu.SemaphoreType.DMA,
            pltpu.SemaphoreType.DMA,
        ),
    )
    def sc_fn(x_hbm, src_hbm, dst_hbm, z_hbm, agg_out,
              sv0, sv1, dv0, dv1, b0, b1, b2, b3, bx, agg_sp,
              g0, g1, g2, g3, s0, s1, s2, s3, i0, i1, i2, i3):
        srcs = (sv0, sv1)
        dsts = (dv0, dv1)
        bufs = (b0, b1, b2, b3)
        gsems = (g0, g1, g2, g3)
        ssems = (s0, s1, s2, s3)
        isrc = (i0, i1)
        idst = (i2, i3)
        c = lax.axis_index("c")
        s = lax.axis_index("s")
        wid = c * NS + s
        r0 = s * ROW_STRIDE
        pltpu.sync_copy(z_hbm, bx)

        @pl.loop(0, NPC)
        def _init(t):
            pltpu.sync_copy(bx, agg_sp.at[pl.ds(r0 + t * PZ, PZ)])

        plsc.subcore_barrier()

        pltpu.async_copy(src_hbm.at[wid, 0], sv0, i0)
        pltpu.async_copy(dst_hbm.at[wid, 0], dv0, i2)

        @pl.loop(0, GRP, step=2)
        def _group(k):
            for p in range(2):
                kk = k + p
                srcv, dstv = srcs[p], dsts[p]
                nsrcv, ndstv = srcs[1 - p], dsts[1 - p]
                pltpu.make_async_copy(src_hbm.at[wid, kk], srcv, isrc[p]).wait()
                pltpu.make_async_copy(dst_hbm.at[wid, kk], dstv, idst[p]).wait()
                gds = []
                for b in range(G):
                    @pl.when(kk >= 1)
                    def _drain_prev():
                        pltpu.make_async_copy(
                            bufs[b], agg_sp.at[dstv.at[b]], ssems[b]).wait()
                    gds.append(pltpu.async_copy(x_hbm.at[srcv.at[b]], bufs[b],
                                                gsems[b]))

                @pl.when(kk + 1 < GRP)
                def _prefetch():
                    pltpu.async_copy(src_hbm.at[wid, kk + 1], nsrcv, isrc[1 - p])
                    pltpu.async_copy(dst_hbm.at[wid, kk + 1], ndstv, idst[1 - p])

                for b in range(G):
                    gds[b].wait()
                    pltpu.async_copy(bufs[b], agg_sp.at[dstv.at[b]],
                                     ssems[b], add=True)

        for b in range(G):
            pltpu.make_async_copy(bufs[b], agg_sp.at[dsts[1].at[b]],
                                  ssems[b]).wait()
        plsc.subcore_barrier()

        @pl.loop(0, NPC)
        def _copyout(t):
            pc = pl.ds(r0 + t * PZ, PZ)
            pltpu.sync_copy(agg_sp.at[pc], bx)
            pltpu.sync_copy(bx, agg_out.at[c, pc])

    return sc_fn(x, src_r, dst_r, zeros_b)


def _sc_degree(dst_r, ones_b, zeros_b):
    mesh = plsc.VectorSubcoreMesh(core_axis_name="c", subcore_axis_name="s")

    @functools.partial(
        pl.kernel,
        out_type=jax.ShapeDtypeStruct((NC, N, D), jnp.float32),
        mesh=mesh,
        scratch_types=(
            pltpu.VMEM((G2, CH2), jnp.int32),
            pltpu.VMEM((CH2, D), jnp.float32),
            pltpu.VMEM((PZ, D), jnp.float32),
            pltpu.VMEM_SHARED((N, D), jnp.float32),
            pltpu.SemaphoreType.DMA,
        ),
    )
    def sc_fn(dst_hbm, ones_hbm, z_hbm, deg_out, dst_v, ones_v, bx, deg_sp,
              ssem):
        c = lax.axis_index("c")
        s = lax.axis_index("s")
        wid = c * NS + s
        r0 = s * ROW_STRIDE
        pltpu.sync_copy(ones_hbm, ones_v)
        pltpu.sync_copy(z_hbm, bx)

        @pl.loop(0, NPC)
        def _init(t):
            pltpu.sync_copy(bx, deg_sp.at[pl.ds(r0 + t * PZ, PZ)])

        plsc.subcore_barrier()

        @pl.loop(0, GRP2)
        def _group(k):
            pltpu.sync_copy(dst_hbm.at[wid, k], dst_v)
            sds = [pltpu.async_copy(ones_v, deg_sp.at[dst_v.at[b]], ssem,
                                    add=True)
                   for b in range(G2)]
            for d in sds:
                d.wait()

        plsc.subcore_barrier()

        @pl.loop(0, NPC)
        def _copyout(t):
            pc = pl.ds(r0 + t * PZ, PZ)
            pltpu.sync_copy(deg_sp.at[pc], bx)
            pltpu.sync_copy(bx, deg_out.at[c, pc])

    return sc_fn(dst_r, ones_b, zeros_b)


BLK = 1000
NBLK = N // BLK


def _tc_stage1(x_ref, agg_ref, deg_ref, wl_ref, bl_ref, wr_ref,
               h_ref, stats_ref):
    x = x_ref[...]
    agg = agg_ref[0] + agg_ref[1]
    deg = deg_ref[0, :, 0] + deg_ref[1, :, 0]
    inv = 1.0 / jnp.maximum(deg, 1.0)
    mean_agg = agg * inv[:, None]
    hi = jax.lax.Precision.HIGHEST
    h = lax.dot_general(mean_agg, wl_ref[...], (((1,), (1,)), ((), ())),
                        precision=hi, preferred_element_type=jnp.float32)
    h = h + lax.dot_general(x, wr_ref[...], (((1,), (1,)), ((), ())),
                            precision=hi, preferred_element_type=jnp.float32)
    h = h + bl_ref[...][None, :] + x
    h = jnp.maximum(h, 0.0)
    h_ref[...] = h
    s1 = jnp.sum(h, axis=0, keepdims=True)
    s2 = jnp.sum(h * h, axis=0, keepdims=True)
    stats_ref[...] = jnp.concatenate([s1, s2, jnp.zeros((6, D), jnp.float32)],
                                     axis=0)[None]


def _tc_stage2(h_ref, stats_ref, gw_ref, gb_ref, gms_ref, out_ref):
    stats = stats_ref[...]
    m = jnp.sum(stats[:, 0, :], axis=0) / N
    q = jnp.sum(stats[:, 1, :], axis=0) / N
    ms = gms_ref[...]
    var = q + m * m * ms * (ms - 2.0)
    scale = gw_ref[...] * lax.rsqrt(var + 1e-5)
    shift = gb_ref[...] - m * ms * scale
    out_ref[...] = h_ref[...] * scale[None, :] + shift[None, :]


def kernel(x, edge_index, W_l, b_l, W_r, gn_weight, gn_bias, gn_mean_scale):
    src_r = edge_index[0].reshape(NW, GRP, G, CH)
    dst_r = edge_index[1].reshape(NW, GRP, G, CH)
    dst_r2 = edge_index[1].reshape(NW, GRP2, G2, CH2)
    zeros_b = jnp.zeros((PZ, D), jnp.float32)
    ones_b = jnp.ones((CH2, D), jnp.float32)
    agg_parts = _sc_gather_scatter(x, src_r, dst_r, zeros_b)
    deg_parts = _sc_degree(dst_r2, ones_b, zeros_b)
    h, stats = pl.pallas_call(
        _tc_stage1,
        grid=(NBLK,),
        in_specs=[
            pl.BlockSpec((BLK, D), lambda i: (i, 0)),
            pl.BlockSpec((NC, BLK, D), lambda i: (0, i, 0)),
            pl.BlockSpec((NC, BLK, D), lambda i: (0, i, 0)),
            pl.BlockSpec((D, D), lambda i: (0, 0)),
            pl.BlockSpec((D,), lambda i: (0,)),
            pl.BlockSpec((D, D), lambda i: (0, 0)),
        ],
        out_specs=[
            pl.BlockSpec((BLK, D), lambda i: (i, 0)),
            pl.BlockSpec((1, 8, D), lambda i: (i, 0, 0)),
        ],
        out_shape=[
            jax.ShapeDtypeStruct((N, D), jnp.float32),
            jax.ShapeDtypeStruct((NBLK, 8, D), jnp.float32),
        ],
    )(x, agg_parts, deg_parts, W_l, b_l, W_r)
    return pl.pallas_call(
        _tc_stage2,
        grid=(NBLK,),
        in_specs=[
            pl.BlockSpec((BLK, D), lambda i: (i, 0)),
            pl.BlockSpec((NBLK, 8, D), lambda i: (0, 0, 0)),
            pl.BlockSpec((D,), lambda i: (0,)),
            pl.BlockSpec((D,), lambda i: (0,)),
            pl.BlockSpec((D,), lambda i: (0,)),
        ],
        out_specs=pl.BlockSpec((BLK, D), lambda i: (i, 0)),
        out_shape=jax.ShapeDtypeStruct((N, D), jnp.float32),
    )(h, stats, gn_weight, gn_bias, gn_mean_scale)


# speedup vs baseline: 8.6408x; 1.0160x over previous
import functools

import jax
import jax.numpy as jnp
from jax import lax
from jax.experimental import pallas as pl
from jax.experimental.pallas import tpu as pltpu
from jax.experimental.pallas import tpu_sc as plsc

N = 10000
D = 128
E = 320000
NC = 2
NS = 16
NW = NC * NS
CH = 50
E_W = E // NW
CHUNKS = E_W // CH
G = 4
GRP = CHUNKS // G
G2 = 10
CH2 = 100
GRP2 = E_W // (G2 * CH2)
ROW_STRIDE = 624
ROWS_T = 640
PZ = 40
NPC = ROWS_T // PZ


def _sc_gather_scatter(x, src_r, dst_r, zeros_b):
    mesh = plsc.VectorSubcoreMesh(core_axis_name="c", subcore_axis_name="s")

    @functools.partial(
        pl.kernel,
        out_type=jax.ShapeDtypeStruct((NC, N, D), jnp.float32),
        mesh=mesh,
        scratch_types=(
            pltpu.VMEM((G, CH), jnp.int32),
            pltpu.VMEM((G, CH), jnp.int32),
            pltpu.VMEM((G, CH), jnp.int32),
            pltpu.VMEM((G, CH), jnp.int32),
            pltpu.VMEM((CH, D), jnp.float32),
            pltpu.VMEM((CH, D), jnp.float32),
            pltpu.VMEM((CH, D), jnp.float32),
            pltpu.VMEM((CH, D), jnp.float32),
            pltpu.VMEM((PZ, D), jnp.float32),
            pltpu.VMEM_SHARED((N, D), jnp.float32),
            pltpu.SemaphoreType.DMA,
            pltpu.SemaphoreType.DMA,
            pltpu.SemaphoreType.DMA,
            pltpu.SemaphoreType.DMA,
            pltpu.SemaphoreType.DMA,
            pltpu.SemaphoreType.DMA,
            pltpu.SemaphoreType.DMA,
            pltpu.SemaphoreType.DMA,
            pltpu.SemaphoreType.DMA,
            pltpu.SemaphoreType.DMA,
            pltpu.SemaphoreType.DMA,
            pltpu.SemaphoreType.DMA,
        ),
    )
    def sc_fn(x_hbm, src_hbm, dst_hbm, z_hbm, agg_out,
              sv0, sv1, dv0, dv1, b0, b1, b2, b3, bx, agg_sp,
              g0, g1, g2, g3, s0, s1, s2, s3, i0, i1, i2, i3):
        srcs = (sv0, sv1)
        dsts = (dv0, dv1)
        bufs = (b0, b1, b2, b3)
        gsems = (g0, g1, g2, g3)
        ssems = (s0, s1, s2, s3)
        isrc = (i0, i1)
        idst = (i2, i3)
        c = lax.axis_index("c")
        s = lax.axis_index("s")
        wid = c * NS + s
        r0 = s * ROW_STRIDE
        pltpu.sync_copy(z_hbm, bx)

        @pl.loop(0, NPC)
        def _init(t):
            pltpu.sync_copy(bx, agg_sp.at[pl.ds(r0 + t * PZ, PZ)])

        plsc.subcore_barrier()

        pltpu.async_copy(src_hbm.at[wid, 0], sv0, i0)
        pltpu.async_copy(dst_hbm.at[wid, 0], dv0, i2)

        @pl.loop(0, GRP, step=2)
        def _group(k):
            for p in range(2):
                kk = k + p
                srcv, dstv = srcs[p], dsts[p]
                nsrcv, ndstv = srcs[1 - p], dsts[1 - p]
                pltpu.make_async_copy(src_hbm.at[wid, kk], srcv, isrc[p]).wait()
                pltpu.make_async_copy(dst_hbm.at[wid, kk], dstv, idst[p]).wait()
                gds = []
                for b in range(G):
                    @pl.when(kk >= 1)
                    def _drain_prev():
                        pltpu.make_async_copy(
                            bufs[b], agg_sp.at[dstv.at[b]], ssems[b]).wait()
                    gds.append(pltpu.async_copy(x_hbm.at[srcv.at[b]], bufs[b],
                                                gsems[b]))

                @pl.when(kk + 1 < GRP)
                def _prefetch():
                    pltpu.async_copy(src_hbm.at[wid, kk + 1], nsrcv, isrc[1 - p])
                    pltpu.async_copy(dst_hbm.at[wid, kk + 1], ndstv, idst[1 - p])

                for b in range(G):
                    gds[b].wait()
                    pltpu.async_copy(bufs[b], agg_sp.at[dstv.at[b]],
                                     ssems[b], add=True)

        for b in range(G):
            pltpu.make_async_copy(bufs[b], agg_sp.at[dsts[1].at[b]],
                                  ssems[b]).wait()
        plsc.subcore_barrier()

        @pl.loop(0, NPC)
        def _copyout(t):
            pc = pl.ds(r0 + t * PZ, PZ)
            pltpu.sync_copy(agg_sp.at[pc], bx)
            pltpu.sync_copy(bx, agg_out.at[c, pc])

    return sc_fn(x, src_r, dst_r, zeros_b)


def _sc_degree(dst_r, ones_b, zeros_b):
    mesh = plsc.VectorSubcoreMesh(core_axis_name="c", subcore_axis_name="s")

    @functools.partial(
        pl.kernel,
        out_type=jax.ShapeDtypeStruct((NC, N, D), jnp.float32),
        mesh=mesh,
        scratch_types=(
            pltpu.VMEM((G2, CH2), jnp.int32),
            pltpu.VMEM((G2, CH2), jnp.int32),
            pltpu.VMEM((CH2, D), jnp.float32),
            pltpu.VMEM((PZ, D), jnp.float32),
            pltpu.VMEM_SHARED((N, D), jnp.float32),
            pltpu.SemaphoreType.DMA,
            pltpu.SemaphoreType.DMA,
            pltpu.SemaphoreType.DMA,
            pltpu.SemaphoreType.DMA,
        ),
    )
    def sc_fn(dst_hbm, ones_hbm, z_hbm, deg_out, dv0, dv1, ones_v, bx,
              deg_sp, ss0, ss1, is0, is1):
        dsts = (dv0, dv1)
        ssem = (ss0, ss1)
        isem = (is0, is1)
        c = lax.axis_index("c")
        s = lax.axis_index("s")
        wid = c * NS + s
        r0 = s * ROW_STRIDE
        pltpu.sync_copy(ones_hbm, ones_v)
        pltpu.sync_copy(z_hbm, bx)

        @pl.loop(0, NPC)
        def _init(t):
            pltpu.sync_copy(bx, deg_sp.at[pl.ds(r0 + t * PZ, PZ)])

        plsc.subcore_barrier()

        pltpu.async_copy(dst_hbm.at[wid, 0], dv0, is0)

        @pl.loop(0, GRP2, step=2)
        def _group(k):
            for p in range(2):
                kk = k + p
                dstv, ndstv = dsts[p], dsts[1 - p]
                pltpu.make_async_copy(dst_hbm.at[wid, kk], dstv, isem[p]).wait()

                @pl.when(kk >= 1)
                def _drain_prev():
                    for b in range(G2):
                        pltpu.make_async_copy(
                            ones_v, deg_sp.at[ndstv.at[b]], ssem[1 - p]).wait()

                @pl.when(kk + 1 < GRP2)
                def _prefetch():
                    pltpu.async_copy(dst_hbm.at[wid, kk + 1], ndstv, isem[1 - p])

                for b in range(G2):
                    pltpu.async_copy(ones_v, deg_sp.at[dstv.at[b]], ssem[p],
                                     add=True)

        for b in range(G2):
            pltpu.make_async_copy(ones_v, deg_sp.at[dsts[1].at[b]],
                                  ssem[1]).wait()
        plsc.subcore_barrier()

        @pl.loop(0, NPC)
        def _copyout(t):
            pc = pl.ds(r0 + t * PZ, PZ)
            pltpu.sync_copy(deg_sp.at[pc], bx)
            pltpu.sync_copy(bx, deg_out.at[c, pc])

    return sc_fn(dst_r, ones_b, zeros_b)


BLK = 1000
NBLK = N // BLK


def _tc_stage1(x_ref, agg_ref, deg_ref, wl_ref, bl_ref, wr_ref,
               h_ref, stats_ref):
    x = x_ref[...]
    agg = agg_ref[0] + agg_ref[1]
    deg = deg_ref[0, :, 0] + deg_ref[1, :, 0]
    inv = 1.0 / jnp.maximum(deg, 1.0)
    mean_agg = agg * inv[:, None]
    hi = jax.lax.Precision.HIGHEST
    h = lax.dot_general(mean_agg, wl_ref[...], (((1,), (1,)), ((), ())),
                        precision=hi, preferred_element_type=jnp.float32)
    h = h + lax.dot_general(x, wr_ref[...], (((1,), (1,)), ((), ())),
                            precision=hi, preferred_element_type=jnp.float32)
    h = h + bl_ref[...][None, :] + x
    h = jnp.maximum(h, 0.0)
    h_ref[...] = h
    s1 = jnp.sum(h, axis=0, keepdims=True)
    s2 = jnp.sum(h * h, axis=0, keepdims=True)
    stats_ref[...] = jnp.concatenate([s1, s2, jnp.zeros((6, D), jnp.float32)],
                                     axis=0)[None]


def _tc_stage2(h_ref, stats_ref, gw_ref, gb_ref, gms_ref, out_ref):
    stats = stats_ref[...]
    m = jnp.sum(stats[:, 0, :], axis=0) / N
    q = jnp.sum(stats[:, 1, :], axis=0) / N
    ms = gms_ref[...]
    var = q + m * m * ms * (ms - 2.0)
    scale = gw_ref[...] * lax.rsqrt(var + 1e-5)
    shift = gb_ref[...] - m * ms * scale
    out_ref[...] = h_ref[...] * scale[None, :] + shift[None, :]


def kernel(x, edge_index, W_l, b_l, W_r, gn_weight, gn_bias, gn_mean_scale):
    src_r = edge_index[0].reshape(NW, GRP, G, CH)
    dst_r = edge_index[1].reshape(NW, GRP, G, CH)
    dst_r2 = edge_index[1].reshape(NW, GRP2, G2, CH2)
    zeros_b = jnp.zeros((PZ, D), jnp.float32)
    ones_b = jnp.ones((CH2, D), jnp.float32)
    agg_parts = _sc_gather_scatter(x, src_r, dst_r, zeros_b)
    deg_parts = _sc_degree(dst_r2, ones_b, zeros_b)
    h, stats = pl.pallas_call(
        _tc_stage1,
        grid=(NBLK,),
        in_specs=[
            pl.BlockSpec((BLK, D), lambda i: (i, 0)),
            pl.BlockSpec((NC, BLK, D), lambda i: (0, i, 0)),
            pl.BlockSpec((NC, BLK, D), lambda i: (0, i, 0)),
            pl.BlockSpec((D, D), lambda i: (0, 0)),
            pl.BlockSpec((D,), lambda i: (0,)),
            pl.BlockSpec((D, D), lambda i: (0, 0)),
        ],
        out_specs=[
            pl.BlockSpec((BLK, D), lambda i: (i, 0)),
            pl.BlockSpec((1, 8, D), lambda i: (i, 0, 0)),
        ],
        out_shape=[
            jax.ShapeDtypeStruct((N, D), jnp.float32),
            jax.ShapeDtypeStruct((NBLK, 8, D), jnp.float32),
        ],
    )(x, agg_parts, deg_parts, W_l, b_l, W_r)
    return pl.pallas_call(
        _tc_stage2,
        grid=(NBLK,),
        in_specs=[
            pl.BlockSpec((BLK, D), lambda i: (i, 0)),
            pl.BlockSpec((NBLK, 8, D), lambda i: (0, 0, 0)),
            pl.BlockSpec((D,), lambda i: (0,)),
            pl.BlockSpec((D,), lambda i: (0,)),
            pl.BlockSpec((D,), lambda i: (0,)),
        ],
        out_specs=pl.BlockSpec((BLK, D), lambda i: (i, 0)),
        out_shape=jax.ShapeDtypeStruct((N, D), jnp.float32),
    )(h, stats, gn_weight, gn_bias, gn_mean_scale)


# speedup vs baseline: 8.8119x; 1.0198x over previous
import functools

import jax
import jax.numpy as jnp
from jax import lax
from jax.experimental import pallas as pl
from jax.experimental.pallas import tpu as pltpu
from jax.experimental.pallas import tpu_sc as plsc

N = 10000
D = 128
E = 320000
NC = 2
NS = 16
NW = NC * NS
CH = 50
E_W = E // NW
CHUNKS = E_W // CH
G = 4
GRP = CHUNKS // G
G2 = 10
CH2 = 100
GRP2 = E_W // (G2 * CH2)
ROW_STRIDE = 624
ROWS_T = 640
PZ = 40
NPC = ROWS_T // PZ


def _sc_gather_scatter(x, src_r, dst_r, zeros_b):
    mesh = plsc.VectorSubcoreMesh(core_axis_name="c", subcore_axis_name="s")

    @functools.partial(
        pl.kernel,
        out_type=(
            jax.ShapeDtypeStruct((NC, N, D), jnp.float32),
            jax.ShapeDtypeStruct((NC, N, D), jnp.float32),
        ),
        mesh=mesh,
        scratch_types=(
            pltpu.VMEM((CH, D), jnp.float32),
            pltpu.VMEM((G, CH), jnp.int32),
            pltpu.VMEM((G, CH), jnp.int32),
            pltpu.VMEM((G, CH), jnp.int32),
            pltpu.VMEM((G, CH), jnp.int32),
            pltpu.VMEM((CH, D), jnp.float32),
            pltpu.VMEM((CH, D), jnp.float32),
            pltpu.VMEM((CH, D), jnp.float32),
            pltpu.VMEM((CH, D), jnp.float32),
            pltpu.VMEM((PZ, D), jnp.float32),
            pltpu.VMEM_SHARED((N, D), jnp.float32),
            pltpu.SemaphoreType.DMA,
            pltpu.SemaphoreType.DMA,
            pltpu.SemaphoreType.DMA,
            pltpu.SemaphoreType.DMA,
            pltpu.SemaphoreType.DMA,
            pltpu.SemaphoreType.DMA,
            pltpu.SemaphoreType.DMA,
            pltpu.SemaphoreType.DMA,
            pltpu.SemaphoreType.DMA,
            pltpu.SemaphoreType.DMA,
            pltpu.SemaphoreType.DMA,
            pltpu.SemaphoreType.DMA,
        ),
    )
    def sc_fn(x_hbm, src_hbm, dst_hbm, z_hbm, ones_hbm, agg_out, deg_out,
              ones_v, sv0, sv1, dv0, dv1, b0, b1, b2, b3, bx, agg_sp,
              g0, g1, g2, g3, s0, s1, s2, s3, i0, i1, i2, i3):
        srcs = (sv0, sv1)
        dsts = (dv0, dv1)
        bufs = (b0, b1, b2, b3)
        gsems = (g0, g1, g2, g3)
        ssems = (s0, s1, s2, s3)
        isrc = (i0, i1)
        idst = (i2, i3)
        c = lax.axis_index("c")
        s = lax.axis_index("s")
        wid = c * NS + s
        r0 = s * ROW_STRIDE
        pltpu.sync_copy(z_hbm, bx)
        pltpu.sync_copy(ones_hbm, ones_v)

        @pl.loop(0, NPC)
        def _init(t):
            pltpu.sync_copy(bx, agg_sp.at[pl.ds(r0 + t * PZ, PZ)])

        plsc.subcore_barrier()

        pltpu.async_copy(src_hbm.at[wid, 0], sv0, i0)
        pltpu.async_copy(dst_hbm.at[wid, 0], dv0, i2)

        @pl.loop(0, GRP, step=2)
        def _group(k):
            for p in range(2):
                kk = k + p
                srcv, dstv = srcs[p], dsts[p]
                nsrcv, ndstv = srcs[1 - p], dsts[1 - p]
                pltpu.make_async_copy(src_hbm.at[wid, kk], srcv, isrc[p]).wait()
                pltpu.make_async_copy(dst_hbm.at[wid, kk], dstv, idst[p]).wait()
                gds = []
                for b in range(G):
                    @pl.when(kk >= 1)
                    def _drain_prev():
                        pltpu.make_async_copy(
                            bufs[b], agg_sp.at[dstv.at[b]], ssems[b]).wait()
                    gds.append(pltpu.async_copy(x_hbm.at[srcv.at[b]], bufs[b],
                                                gsems[b]))

                @pl.when(kk + 1 < GRP)
                def _prefetch():
                    pltpu.async_copy(src_hbm.at[wid, kk + 1], nsrcv, isrc[1 - p])
                    pltpu.async_copy(dst_hbm.at[wid, kk + 1], ndstv, idst[1 - p])

                for b in range(G):
                    gds[b].wait()
                    pltpu.async_copy(bufs[b], agg_sp.at[dstv.at[b]],
                                     ssems[b], add=True)

        for b in range(G):
            pltpu.make_async_copy(bufs[b], agg_sp.at[dsts[1].at[b]],
                                  ssems[b]).wait()
        plsc.subcore_barrier()

        @pl.loop(0, NPC)
        def _copyout(t):
            pc = pl.ds(r0 + t * PZ, PZ)
            pltpu.sync_copy(agg_sp.at[pc], bx)
            pltpu.sync_copy(bx, agg_out.at[c, pc])

        plsc.subcore_barrier()
        pltpu.sync_copy(z_hbm, bx)

        @pl.loop(0, NPC)
        def _init2(t):
            pltpu.sync_copy(bx, agg_sp.at[pl.ds(r0 + t * PZ, PZ)])

        plsc.subcore_barrier()
        pltpu.async_copy(dst_hbm.at[wid, 0], dv0, i2)

        @pl.loop(0, GRP, step=2)
        def _group2(k):
            for p in range(2):
                kk = k + p
                dstv, ndstv = dsts[p], dsts[1 - p]
                pltpu.make_async_copy(dst_hbm.at[wid, kk], dstv, idst[p]).wait()

                @pl.when(kk >= 1)
                def _drain2():
                    for b in range(G):
                        pltpu.make_async_copy(
                            ones_v, agg_sp.at[ndstv.at[b]], ssems[b]).wait()

                @pl.when(kk + 1 < GRP)
                def _prefetch2():
                    pltpu.async_copy(dst_hbm.at[wid, kk + 1], ndstv, idst[1 - p])

                for b in range(G):
                    pltpu.async_copy(ones_v, agg_sp.at[dstv.at[b]], ssems[b],
                                     add=True)

        for b in range(G):
            pltpu.make_async_copy(ones_v, agg_sp.at[dsts[1].at[b]],
                                  ssems[b]).wait()
        plsc.subcore_barrier()

        @pl.loop(0, NPC)
        def _copyout2(t):
            pc = pl.ds(r0 + t * PZ, PZ)
            pltpu.sync_copy(agg_sp.at[pc], bx)
            pltpu.sync_copy(bx, deg_out.at[c, pc])

    return sc_fn(x, src_r, dst_r, zeros_b, jnp.ones((CH, D), jnp.float32))


BLK = 1000
NBLK = N // BLK


def _tc_stage1(x_ref, agg_ref, deg_ref, wl_ref, bl_ref, wr_ref,
               h_ref, stats_ref):
    x = x_ref[...]
    agg = agg_ref[0] + agg_ref[1]
    deg = deg_ref[0, :, 0] + deg_ref[1, :, 0]
    inv = 1.0 / jnp.maximum(deg, 1.0)
    mean_agg = agg * inv[:, None]
    hi = jax.lax.Precision.HIGHEST
    h = lax.dot_general(mean_agg, wl_ref[...], (((1,), (1,)), ((), ())),
                        precision=hi, preferred_element_type=jnp.float32)
    h = h + lax.dot_general(x, wr_ref[...], (((1,), (1,)), ((), ())),
                            precision=hi, preferred_element_type=jnp.float32)
    h = h + bl_ref[...][None, :] + x
    h = jnp.maximum(h, 0.0)
    h_ref[...] = h
    s1 = jnp.sum(h, axis=0, keepdims=True)
    s2 = jnp.sum(h * h, axis=0, keepdims=True)
    stats_ref[...] = jnp.concatenate([s1, s2, jnp.zeros((6, D), jnp.float32)],
                                     axis=0)[None]


def _tc_stage2(h_ref, stats_ref, gw_ref, gb_ref, gms_ref, out_ref):
    stats = stats_ref[...]
    m = jnp.sum(stats[:, 0, :], axis=0) / N
    q = jnp.sum(stats[:, 1, :], axis=0) / N
    ms = gms_ref[...]
    var = q + m * m * ms * (ms - 2.0)
    scale = gw_ref[...] * lax.rsqrt(var + 1e-5)
    shift = gb_ref[...] - m * ms * scale
    out_ref[...] = h_ref[...] * scale[None, :] + shift[None, :]


def kernel(x, edge_index, W_l, b_l, W_r, gn_weight, gn_bias, gn_mean_scale):
    src_r = edge_index[0].reshape(NW, GRP, G, CH)
    dst_r = edge_index[1].reshape(NW, GRP, G, CH)
    zeros_b = jnp.zeros((PZ, D), jnp.float32)
    agg_parts, deg_parts = _sc_gather_scatter(x, src_r, dst_r, zeros_b)
    h, stats = pl.pallas_call(
        _tc_stage1,
        grid=(NBLK,),
        in_specs=[
            pl.BlockSpec((BLK, D), lambda i: (i, 0)),
            pl.BlockSpec((NC, BLK, D), lambda i: (0, i, 0)),
            pl.BlockSpec((NC, BLK, D), lambda i: (0, i, 0)),
            pl.BlockSpec((D, D), lambda i: (0, 0)),
            pl.BlockSpec((D,), lambda i: (0,)),
            pl.BlockSpec((D, D), lambda i: (0, 0)),
        ],
        out_specs=[
            pl.BlockSpec((BLK, D), lambda i: (i, 0)),
            pl.BlockSpec((1, 8, D), lambda i: (i, 0, 0)),
        ],
        out_shape=[
            jax.ShapeDtypeStruct((N, D), jnp.float32),
            jax.ShapeDtypeStruct((NBLK, 8, D), jnp.float32),
        ],
    )(x, agg_parts, deg_parts, W_l, b_l, W_r)
    return pl.pallas_call(
        _tc_stage2,
        grid=(NBLK,),
        in_specs=[
            pl.BlockSpec((BLK, D), lambda i: (i, 0)),
            pl.BlockSpec((NBLK, 8, D), lambda i: (0, 0, 0)),
            pl.BlockSpec((D,), lambda i: (0,)),
            pl.BlockSpec((D,), lambda i: (0,)),
            pl.BlockSpec((D,), lambda i: (0,)),
        ],
        out_specs=pl.BlockSpec((BLK, D), lambda i: (i, 0)),
        out_shape=jax.ShapeDtypeStruct((N, D), jnp.float32),
    )(h, stats, gn_weight, gn_bias, gn_mean_scale)


# speedup vs baseline: 8.9727x; 1.0182x over previous
import functools

import jax
import jax.numpy as jnp
from jax import lax
from jax.experimental import pallas as pl
from jax.experimental.pallas import tpu as pltpu
from jax.experimental.pallas import tpu_sc as plsc

N = 10000
D = 128
E = 320000
NC = 2
NS = 16
NW = NC * NS
CH = 50
E_W = E // NW
CHUNKS = E_W // CH
G = 4
GRP = CHUNKS // G
G2 = 10
CH2 = 100
GRP2 = E_W // (G2 * CH2)
ROW_STRIDE = 624
ROWS_T = 640
PZ = 40
NPC = ROWS_T // PZ


def _sc_gather_scatter(x, src_r, dst_r, zeros_b):
    mesh = plsc.VectorSubcoreMesh(core_axis_name="c", subcore_axis_name="s")

    @functools.partial(
        pl.kernel,
        out_type=(
            jax.ShapeDtypeStruct((NC, N, D), jnp.float32),
            jax.ShapeDtypeStruct((NC, N, D), jnp.float32),
        ),
        mesh=mesh,
        scratch_types=(
            pltpu.VMEM((CH, D), jnp.float32),
            pltpu.VMEM((G, CH), jnp.int32),
            pltpu.VMEM((G, CH), jnp.int32),
            pltpu.VMEM((G, CH), jnp.int32),
            pltpu.VMEM((G, CH), jnp.int32),
            pltpu.VMEM((CH, D), jnp.float32),
            pltpu.VMEM((CH, D), jnp.float32),
            pltpu.VMEM((CH, D), jnp.float32),
            pltpu.VMEM((CH, D), jnp.float32),
            pltpu.VMEM((PZ, D), jnp.float32),
            pltpu.VMEM_SHARED((N, D), jnp.float32),
            pltpu.SemaphoreType.DMA,
            pltpu.SemaphoreType.DMA,
            pltpu.SemaphoreType.DMA,
            pltpu.SemaphoreType.DMA,
            pltpu.SemaphoreType.DMA,
            pltpu.SemaphoreType.DMA,
            pltpu.SemaphoreType.DMA,
            pltpu.SemaphoreType.DMA,
            pltpu.SemaphoreType.DMA,
            pltpu.SemaphoreType.DMA,
            pltpu.SemaphoreType.DMA,
            pltpu.SemaphoreType.DMA,
        ),
    )
    def sc_fn(x_hbm, src_hbm, dst_hbm, z_hbm, ones_hbm, agg_out, deg_out,
              ones_v, sv0, sv1, dv0, dv1, b0, b1, b2, b3, bx, agg_sp,
              g0, g1, g2, g3, s0, s1, s2, s3, i0, i1, i2, i3):
        srcs = (sv0, sv1)
        dsts = (dv0, dv1)
        bufs = (b0, b1, b2, b3)
        gsems = (g0, g1, g2, g3)
        ssems = (s0, s1, s2, s3)
        isrc = (i0, i1)
        idst = (i2, i3)
        c = lax.axis_index("c")
        s = lax.axis_index("s")
        wid = c * NS + s
        r0 = s * ROW_STRIDE
        pltpu.sync_copy(z_hbm, bx)
        pltpu.sync_copy(ones_hbm, ones_v)

        @pl.loop(0, NPC)
        def _init(t):
            pltpu.sync_copy(bx, agg_sp.at[pl.ds(r0 + t * PZ, PZ)])

        plsc.subcore_barrier()

        pltpu.async_copy(src_hbm.at[wid, 0], sv0, i0)
        pltpu.async_copy(dst_hbm.at[wid, 0], dv0, i2)

        @pl.loop(0, GRP, step=2)
        def _group(k):
            for p in range(2):
                kk = k + p
                srcv, dstv = srcs[p], dsts[p]
                nsrcv, ndstv = srcs[1 - p], dsts[1 - p]
                pltpu.make_async_copy(src_hbm.at[wid, kk], srcv, isrc[p]).wait()
                pltpu.make_async_copy(dst_hbm.at[wid, kk], dstv, idst[p]).wait()
                gds = []
                for b in range(G):
                    @pl.when(kk >= 1)
                    def _drain_prev():
                        pltpu.make_async_copy(
                            bufs[b], agg_sp.at[dstv.at[b]], ssems[b]).wait()
                    gds.append(pltpu.async_copy(x_hbm.at[srcv.at[b]], bufs[b],
                                                gsems[b]))

                @pl.when(kk + 1 < GRP)
                def _prefetch():
                    pltpu.async_copy(src_hbm.at[wid, kk + 1], nsrcv, isrc[1 - p])
                    pltpu.async_copy(dst_hbm.at[wid, kk + 1], ndstv, idst[1 - p])

                for b in range(G):
                    gds[b].wait()
                    pltpu.async_copy(bufs[b], agg_sp.at[dstv.at[b]],
                                     ssems[b], add=True)

        for b in range(G):
            pltpu.make_async_copy(bufs[b], agg_sp.at[dsts[1].at[b]],
                                  ssems[b]).wait()
        plsc.subcore_barrier()

        @pl.loop(0, NPC)
        def _copyout(t):
            pc = pl.ds(r0 + t * PZ, PZ)
            pltpu.sync_copy(agg_sp.at[pc], bx)
            pltpu.sync_copy(bx, agg_out.at[c, pc])

        plsc.subcore_barrier()
        pltpu.sync_copy(z_hbm, bx)

        @pl.loop(0, NPC)
        def _init2(t):
            pltpu.sync_copy(bx, agg_sp.at[pl.ds(r0 + t * PZ, PZ)])

        plsc.subcore_barrier()
        pltpu.async_copy(dst_hbm.at[wid, 0], dv0, i2)

        @pl.loop(0, GRP, step=2)
        def _group2(k):
            for p in range(2):
                kk = k + p
                dstv, ndstv = dsts[p], dsts[1 - p]
                pltpu.make_async_copy(dst_hbm.at[wid, kk], dstv, idst[p]).wait()

                @pl.when(kk >= 1)
                def _drain2():
                    for b in range(G):
                        pltpu.make_async_copy(
                            ones_v, agg_sp.at[ndstv.at[b]], ssems[b]).wait()

                @pl.when(kk + 1 < GRP)
                def _prefetch2():
                    pltpu.async_copy(dst_hbm.at[wid, kk + 1], ndstv, idst[1 - p])

                for b in range(G):
                    pltpu.async_copy(ones_v, agg_sp.at[dstv.at[b]], ssems[b],
                                     add=True)

        for b in range(G):
            pltpu.make_async_copy(ones_v, agg_sp.at[dsts[1].at[b]],
                                  ssems[b]).wait()
        plsc.subcore_barrier()

        @pl.loop(0, NPC)
        def _copyout2(t):
            pc = pl.ds(r0 + t * PZ, PZ)
            pltpu.sync_copy(agg_sp.at[pc], bx)
            pltpu.sync_copy(bx, deg_out.at[c, pc])

    return sc_fn(x, src_r, dst_r, zeros_b, jnp.ones((CH, D), jnp.float32))


BLK = 1000
NBLK = N // BLK


def _tc_fused(x_ref, agg_ref, deg_ref, wl_ref, bl_ref, wr_ref,
              gw_ref, gb_ref, gms_ref, out_ref, h_s, st_s):
    p = pl.program_id(0)
    i = pl.program_id(1)

    @pl.when(p == 0)
    def _pass1():
        x = x_ref[...]
        agg = agg_ref[0] + agg_ref[1]
        deg = deg_ref[0, :, 0] + deg_ref[1, :, 0]
        inv = 1.0 / jnp.maximum(deg, 1.0)
        mean_agg = agg * inv[:, None]
        hi = jax.lax.Precision.HIGHEST
        h = lax.dot_general(mean_agg, wl_ref[...], (((1,), (1,)), ((), ())),
                            precision=hi, preferred_element_type=jnp.float32)
        h = h + lax.dot_general(x, wr_ref[...], (((1,), (1,)), ((), ())),
                                precision=hi, preferred_element_type=jnp.float32)
        h = h + bl_ref[...][None, :] + x
        h = jnp.maximum(h, 0.0)
        h_s[pl.ds(i * BLK, BLK), :] = h
        s1 = jnp.sum(h, axis=0, keepdims=True)
        s2 = jnp.sum(h * h, axis=0, keepdims=True)

        @pl.when(i == 0)
        def _set():
            st_s[0:1, :] = s1
            st_s[1:2, :] = s2

        @pl.when(i > 0)
        def _acc():
            st_s[0:1, :] = st_s[0:1, :] + s1
            st_s[1:2, :] = st_s[1:2, :] + s2

    @pl.when(p == 1)
    def _pass2():
        h = h_s[pl.ds(i * BLK, BLK), :]
        m = st_s[0, :] / N
        q = st_s[1, :] / N
        ms = gms_ref[...]
        var = q + m * m * ms * (ms - 2.0)
        scale = gw_ref[...] * lax.rsqrt(var + 1e-5)
        shift = gb_ref[...] - m * ms * scale
        out_ref[...] = h * scale[None, :] + shift[None, :]


def kernel(x, edge_index, W_l, b_l, W_r, gn_weight, gn_bias, gn_mean_scale):
    src_r = edge_index[0].reshape(NW, GRP, G, CH)
    dst_r = edge_index[1].reshape(NW, GRP, G, CH)
    zeros_b = jnp.zeros((PZ, D), jnp.float32)
    agg_parts, deg_parts = _sc_gather_scatter(x, src_r, dst_r, zeros_b)
    def _bi(p, i):
        return i * (1 - p) + (NBLK - 1) * p

    return pl.pallas_call(
        _tc_fused,
        grid=(2, NBLK),
        in_specs=[
            pl.BlockSpec((BLK, D), lambda p, i: (_bi(p, i), 0)),
            pl.BlockSpec((NC, BLK, D), lambda p, i: (0, _bi(p, i), 0)),
            pl.BlockSpec((NC, BLK, D), lambda p, i: (0, _bi(p, i), 0)),
            pl.BlockSpec((D, D), lambda p, i: (0, 0)),
            pl.BlockSpec((D,), lambda p, i: (0,)),
            pl.BlockSpec((D, D), lambda p, i: (0, 0)),
            pl.BlockSpec((D,), lambda p, i: (0,)),
            pl.BlockSpec((D,), lambda p, i: (0,)),
            pl.BlockSpec((D,), lambda p, i: (0,)),
        ],
        out_specs=pl.BlockSpec((BLK, D), lambda p, i: (i, 0)),
        out_shape=jax.ShapeDtypeStruct((N, D), jnp.float32),
        scratch_shapes=[
            pltpu.VMEM((N, D), jnp.float32),
            pltpu.VMEM((8, D), jnp.float32),
        ],
    )(x, agg_parts, deg_parts, W_l, b_l, W_r, gn_weight, gn_bias,
      gn_mean_scale)


# speedup vs baseline: 8.9775x; 1.0005x over previous
import functools

import jax
import jax.numpy as jnp
from jax import lax
from jax.experimental import pallas as pl
from jax.experimental.pallas import tpu as pltpu
from jax.experimental.pallas import tpu_sc as plsc

N = 10000
D = 128
E = 320000
NC = 2
NS = 16
NW = NC * NS
CH = 50
E_W = E // NW
CHUNKS = E_W // CH
G = 4
GRP = CHUNKS // G
ROW_STRIDE = 624
ROWS_T = 640
PZ = 40
NPC = ROWS_T // PZ


def _sc_gather_scatter(x, src_r, dst_r, zeros_b):
    mesh = plsc.VectorSubcoreMesh(core_axis_name="c", subcore_axis_name="s")

    @functools.partial(
        pl.kernel,
        out_type=(
            jax.ShapeDtypeStruct((NC, N, D), jnp.float32),
            jax.ShapeDtypeStruct((NC, N, D), jnp.float32),
        ),
        mesh=mesh,
        scratch_types=(
            pltpu.VMEM((CH, D), jnp.float32),
            pltpu.VMEM((G, CH), jnp.int32),
            pltpu.VMEM((G, CH), jnp.int32),
            pltpu.VMEM((G, CH), jnp.int32),
            pltpu.VMEM((G, CH), jnp.int32),
            pltpu.VMEM((CH, D), jnp.float32),
            pltpu.VMEM((CH, D), jnp.float32),
            pltpu.VMEM((CH, D), jnp.float32),
            pltpu.VMEM((CH, D), jnp.float32),
            pltpu.VMEM((PZ, D), jnp.float32),
            pltpu.VMEM_SHARED((N, D), jnp.float32),
            pltpu.SemaphoreType.DMA,
            pltpu.SemaphoreType.DMA,
            pltpu.SemaphoreType.DMA,
            pltpu.SemaphoreType.DMA,
            pltpu.SemaphoreType.DMA,
            pltpu.SemaphoreType.DMA,
            pltpu.SemaphoreType.DMA,
            pltpu.SemaphoreType.DMA,
            pltpu.SemaphoreType.DMA,
            pltpu.SemaphoreType.DMA,
            pltpu.SemaphoreType.DMA,
            pltpu.SemaphoreType.DMA,
        ),
    )
    def sc_fn(x_hbm, src_hbm, dst_hbm, z_hbm, ones_hbm, agg_out, deg_out,
              ones_v, sv0, sv1, dv0, dv1, b0, b1, b2, b3, bx, agg_sp,
              g0, g1, g2, g3, s0, s1, s2, s3, i0, i1, i2, i3):
        srcs = (sv0, sv1)
        dsts = (dv0, dv1)
        bufs = (b0, b1, b2, b3)
        gsems = (g0, g1, g2, g3)
        ssems = (s0, s1, s2, s3)
        isrc = (i0, i1)
        idst = (i2, i3)
        c = lax.axis_index("c")
        s = lax.axis_index("s")
        wid = c * NS + s
        r0 = s * ROW_STRIDE
        pltpu.sync_copy(z_hbm, bx)
        pltpu.sync_copy(ones_hbm, ones_v)

        @pl.loop(0, NPC)
        def _init(t):
            pltpu.sync_copy(bx, agg_sp.at[pl.ds(r0 + t * PZ, PZ)])

        plsc.subcore_barrier()

        pltpu.async_copy(src_hbm.at[wid, 0], sv0, i0)
        pltpu.async_copy(dst_hbm.at[wid, 0], dv0, i2)

        @pl.loop(0, GRP, step=2)
        def _group(k):
            for p in range(2):
                kk = k + p
                srcv, dstv = srcs[p], dsts[p]
                nsrcv, ndstv = srcs[1 - p], dsts[1 - p]
                pltpu.make_async_copy(src_hbm.at[wid, kk], srcv, isrc[p]).wait()
                pltpu.make_async_copy(dst_hbm.at[wid, kk], dstv, idst[p]).wait()
                gds = []
                for b in range(G):
                    @pl.when(kk >= 1)
                    def _drain_prev():
                        pltpu.make_async_copy(
                            bufs[b], agg_sp.at[dstv.at[b]], ssems[b]).wait()
                    gds.append(pltpu.async_copy(x_hbm.at[srcv.at[b]], bufs[b],
                                                gsems[b]))

                @pl.when(kk + 1 < GRP)
                def _prefetch():
                    pltpu.async_copy(src_hbm.at[wid, kk + 1], nsrcv, isrc[1 - p])
                    pltpu.async_copy(dst_hbm.at[wid, kk + 1], ndstv, idst[1 - p])

                for b in range(G):
                    gds[b].wait()
                    pltpu.async_copy(bufs[b], agg_sp.at[dstv.at[b]],
                                     ssems[b], add=True)

        for b in range(G):
            pltpu.make_async_copy(bufs[b], agg_sp.at[dsts[1].at[b]],
                                  ssems[b]).wait()
        plsc.subcore_barrier()

        @pl.loop(0, NPC)
        def _copyout(t):
            pc = pl.ds(r0 + t * PZ, PZ)
            pltpu.sync_copy(agg_sp.at[pc], bx)
            pltpu.sync_copy(bx, agg_out.at[c, pc])

        plsc.subcore_barrier()
        pltpu.sync_copy(z_hbm, bx)

        @pl.loop(0, NPC)
        def _init2(t):
            pltpu.sync_copy(bx, agg_sp.at[pl.ds(r0 + t * PZ, PZ)])

        plsc.subcore_barrier()
        pltpu.async_copy(dst_hbm.at[wid, 0], dv0, i2)

        @pl.loop(0, GRP, step=2)
        def _group2(k):
            for p in range(2):
                kk = k + p
                dstv, ndstv = dsts[p], dsts[1 - p]
                pltpu.make_async_copy(dst_hbm.at[wid, kk], dstv, idst[p]).wait()

                @pl.when(kk >= 1)
                def _drain2():
                    for b in range(G):
                        pltpu.make_async_copy(
                            ones_v, agg_sp.at[ndstv.at[b]], ssems[b]).wait()

                @pl.when(kk + 1 < GRP)
                def _prefetch2():
                    pltpu.async_copy(dst_hbm.at[wid, kk + 1], ndstv, idst[1 - p])

                for b in range(G):
                    pltpu.async_copy(ones_v, agg_sp.at[dstv.at[b]], ssems[b],
                                     add=True)

        for b in range(G):
            pltpu.make_async_copy(ones_v, agg_sp.at[dsts[1].at[b]],
                                  ssems[b]).wait()
        plsc.subcore_barrier()

        @pl.loop(0, NPC)
        def _copyout2(t):
            pc = pl.ds(r0 + t * PZ, PZ)
            pltpu.sync_copy(agg_sp.at[pc], bx)
            pltpu.sync_copy(bx, deg_out.at[c, pc])

    return sc_fn(x, src_r, dst_r, zeros_b, jnp.ones((CH, D), jnp.float32))


BLK = 1000
NBLK = N // BLK


def _tc_fused(x_ref, agg_ref, deg_ref, wl_ref, bl_ref, wr_ref,
              gw_ref, gb_ref, gms_ref, out_ref, h_s, st_s):
    p = pl.program_id(0)
    i = pl.program_id(1)

    @pl.when(p == 0)
    def _pass1():
        x = x_ref[...]
        agg = agg_ref[0] + agg_ref[1]
        deg = deg_ref[0, :, 0] + deg_ref[1, :, 0]
        inv = 1.0 / jnp.maximum(deg, 1.0)
        mean_agg = agg * inv[:, None]
        hi = jax.lax.Precision.HIGHEST
        h = lax.dot_general(mean_agg, wl_ref[...], (((1,), (1,)), ((), ())),
                            precision=hi, preferred_element_type=jnp.float32)
        h = h + lax.dot_general(x, wr_ref[...], (((1,), (1,)), ((), ())),
                                precision=hi, preferred_element_type=jnp.float32)
        h = h + bl_ref[...][None, :] + x
        h = jnp.maximum(h, 0.0)
        h_s[pl.ds(i * BLK, BLK), :] = h
        s1 = jnp.sum(h, axis=0, keepdims=True)
        s2 = jnp.sum(h * h, axis=0, keepdims=True)

        @pl.when(i == 0)
        def _set():
            st_s[0:1, :] = s1
            st_s[1:2, :] = s2

        @pl.when(i > 0)
        def _acc():
            st_s[0:1, :] = st_s[0:1, :] + s1
            st_s[1:2, :] = st_s[1:2, :] + s2

    @pl.when(p == 1)
    def _pass2():
        h = h_s[pl.ds(i * BLK, BLK), :]
        m = st_s[0, :] / N
        q = st_s[1, :] / N
        ms = gms_ref[...]
        var = q + m * m * ms * (ms - 2.0)
        scale = gw_ref[...] * lax.rsqrt(var + 1e-5)
        shift = gb_ref[...] - m * ms * scale
        out_ref[...] = h * scale[None, :] + shift[None, :]


def kernel(x, edge_index, W_l, b_l, W_r, gn_weight, gn_bias, gn_mean_scale):
    src_r = edge_index[0].reshape(NW, GRP, G, CH)
    dst_r = edge_index[1].reshape(NW, GRP, G, CH)
    zeros_b = jnp.zeros((PZ, D), jnp.float32)
    agg_parts, deg_parts = _sc_gather_scatter(x, src_r, dst_r, zeros_b)
    def _bi(p, i):
        return i * (1 - p) + (NBLK - 1) * p

    return pl.pallas_call(
        _tc_fused,
        grid=(2, NBLK),
        in_specs=[
            pl.BlockSpec((BLK, D), lambda p, i: (_bi(p, i), 0)),
            pl.BlockSpec((NC, BLK, D), lambda p, i: (0, _bi(p, i), 0)),
            pl.BlockSpec((NC, BLK, D), lambda p, i: (0, _bi(p, i), 0)),
            pl.BlockSpec((D, D), lambda p, i: (0, 0)),
            pl.BlockSpec((D,), lambda p, i: (0,)),
            pl.BlockSpec((D, D), lambda p, i: (0, 0)),
            pl.BlockSpec((D,), lambda p, i: (0,)),
            pl.BlockSpec((D,), lambda p, i: (0,)),
            pl.BlockSpec((D,), lambda p, i: (0,)),
        ],
        out_specs=pl.BlockSpec((BLK, D), lambda p, i: (i, 0)),
        out_shape=jax.ShapeDtypeStruct((N, D), jnp.float32),
        scratch_shapes=[
            pltpu.VMEM((N, D), jnp.float32),
            pltpu.VMEM((8, D), jnp.float32),
        ],
    )(x, agg_parts, deg_parts, W_l, b_l, W_r, gn_weight, gn_bias,
      gn_mean_scale)
